# baseline (speedup 1.0000x reference)
import functools

import jax
import jax.numpy as jnp
import numpy as np
from jax import lax
from jax.experimental import pallas as pl
from jax.experimental.pallas import tpu as pltpu
from jax.experimental.pallas import tpu_sc as plsc

CODEBOOK_SIZE = 256
D = 1024
WORD_LEN = 3
B = 16
T = 3072
NUM_WORDS = (T // WORD_LEN) * B

NC = 2
NS = 16
NW = NC * NS

NHALF = 2
NG = NW // NHALF
WPW = NUM_WORDS // NG
WCH = 32
NCHUNK = WPW // WCH
NBUF = 2
LANES = 16
DP = D // 2
DPH = DP // NHALF
DH = D // NHALF


def _sc_body(idx_hbm, table_hbm, out_hbm, idx_v, table_v, out_v, wsems):
  wid = lax.axis_index("s") * NC + lax.axis_index("c")
  g = wid // NHALF
  half = lax.rem(wid, NHALF)
  word_base = g * WPW

  pltpu.sync_copy(table_hbm.at[half], table_v)
  pltpu.sync_copy(idx_hbm.at[:, pl.ds(word_base, WPW)], idx_v)

  def start_write(c, buf):
    pltpu.async_copy(
        out_v.at[buf],
        out_hbm.at[pl.ds(word_base + c * WCH, WCH), pl.ds(half * DH, DH)],
        wsems.at[buf])

  def wait_write(c, buf):
    pltpu.make_async_copy(
        out_v.at[buf],
        out_hbm.at[pl.ds(word_base + c * WCH, WCH), pl.ds(half * DH, DH)],
        wsems.at[buf]).wait()

  third = jnp.float32(1.0 / 3.0)
  himask = jnp.int32(-65536)
  iota = lax.iota(jnp.int32, LANES)
  iota8 = lax.shift_left(iota, 3)

  def splat(v, lanes):
    return lax.gather(
        v, lanes[:, None],
        lax.GatherDimensionNumbers(offset_dims=(), collapsed_slice_dims=(0,),
                                   start_index_map=(0,)),
        (1,), mode=lax.GatherScatterMode.PROMISE_IN_BOUNDS)

  def unpack2(x):
    lo = lax.bitcast_convert_type(lax.shift_left(x, 16), jnp.float32)
    hi = lax.bitcast_convert_type(lax.bitwise_and(x, himask), jnp.float32)
    return lo, hi

  def compute(c, buf):
    def word_body(w, carry):
      lane = lax.bitwise_and(c * WCH + w, LANES - 1)
      lanes = lax.broadcast_in_dim(lane, (LANES,), ())
      vbase = (c * WCH + w) // LANES * LANES
      ia = idx_v[0, pl.ds(vbase, LANES)]
      ib = idx_v[1, pl.ds(vbase, LANES)]
      ic = idx_v[2, pl.ds(vbase, LANES)]
      ra = lax.shift_left(splat(ia, lanes), 8) + iota8
      rb = lax.shift_left(splat(ib, lanes), 8) + iota8
      rc = lax.shift_left(splat(ic, lanes), 8) + iota8
      for j in range(DPH // LANES):
        cj = (j & 7) + (j >> 3) * 128
        a = plsc.load_gather(table_v, [ra + cj])
        b = plsc.load_gather(table_v, [rb + cj])
        cc = plsc.load_gather(table_v, [rc + cj])
        la, ha = unpack2(a)
        lb, hb = unpack2(b)
        lc, hc = unpack2(cc)
        out_v[buf, w, pl.ds(j * 2 * LANES, LANES)] = (la + lb + lc) * third
        out_v[buf, w, pl.ds((j * 2 + 1) * LANES, LANES)] = (ha + hb + hc) * third
      return carry

    lax.fori_loop(0, WCH, word_body, 0, unroll=False)

  def chunk_body(c, carry):
    buf = lax.rem(c, NBUF)
    @pl.when(c >= NBUF)
    def _():
      wait_write(c - NBUF, buf)
    compute(c, buf)
    start_write(c, buf)
    return carry

  lax.fori_loop(0, NCHUNK, chunk_body, 0, unroll=False)
  wait_write(NCHUNK - 2, lax.rem(NCHUNK - 2, NBUF))
  wait_write(NCHUNK - 1, lax.rem(NCHUNK - 1, NBUF))


@jax.jit
def _compose_words(idx3, table_packed):
  mesh = plsc.VectorSubcoreMesh(core_axis_name="c", subcore_axis_name="s")
  run = pl.kernel(
      _sc_body,
      out_type=jax.ShapeDtypeStruct((NUM_WORDS, D), jnp.float32),
      mesh=mesh,
      compiler_params=pltpu.CompilerParams(use_tc_tiling_on_sc=False, needs_layout_passes=False),
      scratch_types=[
          pltpu.VMEM((WORD_LEN, WPW), jnp.int32),
          pltpu.VMEM((CODEBOOK_SIZE * DPH,), jnp.int32),
          pltpu.VMEM((NBUF, WCH, DH), jnp.float32),
          pltpu.SemaphoreType.DMA((NBUF,)),
      ],
  )
  return run(idx3, table_packed)


def _pack_table(table):
  tb = table.astype(jnp.bfloat16).reshape(CODEBOOK_SIZE, D // 32, 2, 16)
  tb = jnp.transpose(tb, (0, 1, 3, 2))
  packed = lax.bitcast_convert_type(tb, jnp.int32).reshape(CODEBOOK_SIZE, DP)
  halves = jnp.transpose(packed.reshape(CODEBOOK_SIZE, NHALF, DPH),
                         (1, 0, 2))
  off = np.arange(DPH)
  perm_src = 16 * ((off // 128) * 8 + off % 8) + (off % 128) // 8
  halves = halves[:, :, perm_src]
  return halves.reshape(NHALF, CODEBOOK_SIZE * DPH)


def kernel(char_indices, char_codebook):
  idx3 = jnp.transpose(
      jnp.reshape(char_indices.astype(jnp.int32), (NUM_WORDS, WORD_LEN)))
  words = _compose_words(idx3, _pack_table(char_codebook))
  return jnp.reshape(words, (B, NUM_WORDS // B, D))


# speedup vs baseline: 2.1208x; 2.1208x over previous
import functools

import jax
import jax.numpy as jnp
from jax import lax
from jax.experimental import pallas as pl
from jax.experimental.pallas import tpu as pltpu
from jax.experimental.pallas import tpu_sc as plsc

CODEBOOK_SIZE = 256
D = 1024
WORD_LEN = 3
B = 16
T = 3072
NUM_WORDS = (T // WORD_LEN) * B

NC = 2
NS = 16
NW = NC * NS

WPW = NUM_WORDS // NW
WCH = 16
NCHUNK = WPW // WCH
NBUF = 2
LANES = 16
ROWS = WCH * WORD_LEN
DP = D // 2


def _sc_body(idx_hbm, table_hbm, out_hbm, idx_v, rows_v, out_v, gsems, wsems):
  wid = lax.axis_index("s") * NC + lax.axis_index("c")
  word_base = wid * WPW

  pltpu.sync_copy(idx_hbm.at[pl.ds(word_base * WORD_LEN, WPW * WORD_LEN)],
                  idx_v)

  def start_gather(c, buf):
    idx_sl = idx_v.at[pl.ds(c * ROWS, ROWS)]
    pltpu.async_copy(table_hbm.at[idx_sl], rows_v.at[buf], gsems.at[buf])

  def wait_gather(buf):
    pltpu.make_async_copy(table_hbm.at[idx_v.at[pl.ds(0, ROWS)]],
                          rows_v.at[buf], gsems.at[buf]).wait()

  def start_write(c, buf):
    pltpu.async_copy(out_v.at[buf],
                     out_hbm.at[pl.ds(word_base + c * WCH, WCH)],
                     wsems.at[buf])

  def wait_write(c, buf):
    pltpu.make_async_copy(out_v.at[buf],
                          out_hbm.at[pl.ds(word_base + c * WCH, WCH)],
                          wsems.at[buf]).wait()

  third = jnp.float32(1.0 / 3.0)
  himask = jnp.int32(-65536)

  def unpack2(x):
    lo = lax.bitcast_convert_type(lax.shift_left(x, 16), jnp.float32)
    hi = lax.bitcast_convert_type(x, jnp.float32)
    return lo, hi

  def compute(buf):
    def word_body(w, carry):
      r = 3 * w
      for j in range(DP // LANES):
        sl = pl.ds(j * LANES, LANES)
        la, ha = unpack2(rows_v[buf, r, sl])
        lb, hb = unpack2(rows_v[buf, r + 1, sl])
        lc, hc = unpack2(rows_v[buf, r + 2, sl])
        out_v[buf, w, pl.ds(j * 2 * LANES, LANES)] = (la + lb + lc) * third
        out_v[buf, w, pl.ds((j * 2 + 1) * LANES, LANES)] = (ha + hb + hc) * third
      return carry

    lax.fori_loop(0, WCH, word_body, 0, unroll=False)

  start_gather(0, 0)
  start_gather(1, 1)

  def chunk_body(c, carry):
    buf = lax.rem(c, NBUF)
    wait_gather(buf)
    @pl.when(c >= NBUF)
    def _():
      wait_write(c - NBUF, buf)
    compute(buf)
    start_write(c, buf)
    @pl.when(c + NBUF < NCHUNK)
    def _():
      start_gather(c + NBUF, buf)
    return carry

  lax.fori_loop(0, NCHUNK, chunk_body, 0, unroll=False)
  wait_write(NCHUNK - 2, lax.rem(NCHUNK - 2, NBUF))
  wait_write(NCHUNK - 1, lax.rem(NCHUNK - 1, NBUF))


@jax.jit
def _compose_words(idx_flat, table_packed):
  mesh = plsc.VectorSubcoreMesh(core_axis_name="c", subcore_axis_name="s")
  run = pl.kernel(
      _sc_body,
      out_type=jax.ShapeDtypeStruct((NUM_WORDS, D), jnp.float32),
      mesh=mesh,
      scratch_types=[
          pltpu.VMEM((WPW * WORD_LEN,), jnp.int32),
          pltpu.VMEM((NBUF, ROWS, DP), jnp.int32),
          pltpu.VMEM((NBUF, WCH, D), jnp.float32),
          pltpu.SemaphoreType.DMA((NBUF,)),
          pltpu.SemaphoreType.DMA((NBUF,)),
      ],
  )
  return run(idx_flat, table_packed)


def _pack_table(table):
  tb = table.astype(jnp.bfloat16).reshape(CODEBOOK_SIZE, D // 32, 2, 16)
  tb = jnp.transpose(tb, (0, 1, 3, 2))
  return lax.bitcast_convert_type(tb, jnp.int32).reshape(CODEBOOK_SIZE, DP)


def kernel(char_indices, char_codebook):
  idx_flat = jnp.reshape(char_indices.astype(jnp.int32), (-1,))
  words = _compose_words(idx_flat, _pack_table(char_codebook))
  return jnp.reshape(words, (B, NUM_WORDS // B, D))


# speedup vs baseline: 2.3935x; 1.1286x over previous
import functools

import jax
import jax.numpy as jnp
from jax import lax
from jax.experimental import pallas as pl
from jax.experimental.pallas import tpu as pltpu
from jax.experimental.pallas import tpu_sc as plsc

CODEBOOK_SIZE = 256
D = 1024
WORD_LEN = 3
B = 16
T = 3072
NUM_WORDS = (T // WORD_LEN) * B

NC = 2
NS = 16
NW = NC * NS

WPW = NUM_WORDS // NW
WCH = 16
NCHUNK = WPW // WCH
NBUF = 2
LANES = 16
ROWS = WCH * WORD_LEN
DP = D // 2


def _sc_body(idx_hbm, table_hbm, out_hbm, idx_v, rows_v, out_v, gsems, wsems):
  wid = lax.axis_index("s") * NC + lax.axis_index("c")
  word_base = wid * WPW

  pltpu.sync_copy(idx_hbm.at[pl.ds(word_base * WORD_LEN, WPW * WORD_LEN)],
                  idx_v)

  def start_gather(c, buf):
    idx_sl = idx_v.at[pl.ds(c * ROWS, ROWS)]
    pltpu.async_copy(table_hbm.at[idx_sl], rows_v.at[buf], gsems.at[buf])

  def wait_gather(buf):
    pltpu.make_async_copy(table_hbm.at[idx_v.at[pl.ds(0, ROWS)]],
                          rows_v.at[buf], gsems.at[buf]).wait()

  def start_write(c, buf):
    pltpu.async_copy(out_v.at[buf],
                     out_hbm.at[pl.ds(word_base + c * WCH, WCH)],
                     wsems.at[buf])

  def wait_write(c, buf):
    pltpu.make_async_copy(out_v.at[buf],
                          out_hbm.at[pl.ds(word_base + c * WCH, WCH)],
                          wsems.at[buf]).wait()

  def compute(buf):
    def word_body(w, carry):
      r = 3 * w
      for j in range(DP // LANES):
        sl = pl.ds(j * LANES, LANES)
        a = plsc.bitcast(rows_v[buf, r, sl], jnp.bfloat16)
        b = plsc.bitcast(rows_v[buf, r + 1, sl], jnp.bfloat16)
        cc = plsc.bitcast(rows_v[buf, r + 2, sl], jnp.bfloat16)
        s = plsc.bitcast((a + b) + cc, jnp.int32)
        lo = lax.bitcast_convert_type(lax.shift_left(s, 16), jnp.float32)
        hi = lax.bitcast_convert_type(s, jnp.float32)
        out_v[buf, w, pl.ds(j * 2 * LANES, LANES)] = lo
        out_v[buf, w, pl.ds((j * 2 + 1) * LANES, LANES)] = hi
      return carry

    lax.fori_loop(0, WCH, word_body, 0, unroll=False)

  start_gather(0, 0)
  start_gather(1, 1)

  def chunk_body(c, carry):
    buf = lax.rem(c, NBUF)
    wait_gather(buf)
    @pl.when(c >= NBUF)
    def _():
      wait_write(c - NBUF, buf)
    compute(buf)
    start_write(c, buf)
    @pl.when(c + NBUF < NCHUNK)
    def _():
      start_gather(c + NBUF, buf)
    return carry

  lax.fori_loop(0, NCHUNK, chunk_body, 0, unroll=False)
  wait_write(NCHUNK - 2, lax.rem(NCHUNK - 2, NBUF))
  wait_write(NCHUNK - 1, lax.rem(NCHUNK - 1, NBUF))


@jax.jit
def _compose_words(idx_flat, table_packed):
  mesh = plsc.VectorSubcoreMesh(core_axis_name="c", subcore_axis_name="s")
  run = pl.kernel(
      _sc_body,
      out_type=jax.ShapeDtypeStruct((NUM_WORDS, D), jnp.float32),
      mesh=mesh,
      compiler_params=pltpu.CompilerParams(needs_layout_passes=False),
      scratch_types=[
          pltpu.VMEM((WPW * WORD_LEN,), jnp.int32),
          pltpu.VMEM((NBUF, ROWS, DP), jnp.int32),
          pltpu.VMEM((NBUF, WCH, D), jnp.float32),
          pltpu.SemaphoreType.DMA((NBUF,)),
          pltpu.SemaphoreType.DMA((NBUF,)),
      ],
  )
  return run(idx_flat, table_packed)


def _pack_table(table):
  tb = (table * (1.0 / 3.0)).astype(jnp.bfloat16)
  tb = tb.reshape(CODEBOOK_SIZE, D // 32, 2, 16)
  tb = jnp.transpose(tb, (0, 1, 3, 2))
  return lax.bitcast_convert_type(tb, jnp.int32).reshape(CODEBOOK_SIZE, DP)


def kernel(char_indices, char_codebook):
  idx_flat = jnp.reshape(char_indices.astype(jnp.int32), (-1,))
  words = _compose_words(idx_flat, _pack_table(char_codebook))
  return jnp.reshape(words, (B, NUM_WORDS // B, D))
